# baseline (speedup 1.0000x reference)
import functools

import jax
import jax.numpy as jnp
from jax import lax
from jax.experimental import pallas as pl
from jax.experimental.pallas import tpu as pltpu
from jax.experimental.pallas import tpu_sc as plsc

N_NODES = 10000
D = 128
E = 320000
NC = 2
NS = 16
NW = NC * NS
E_PER_W = E // NW
CHUNK = 80
N_CHUNKS = E_PER_W // CHUNK
NB = 4
WB_ROWS = 624
TAIL_ROWS = N_NODES - NS * WB_ROWS
ZROWS = 16
L = 16


def _sc_segment_sum(x, src1, dst1, dt1, norm1, lam16):
    mesh = plsc.VectorSubcoreMesh(core_axis_name="c", subcore_axis_name="s")

    @functools.partial(
        pl.kernel,
        out_type=jax.ShapeDtypeStruct((NC, N_NODES, D), jnp.float32),
        mesh=mesh,
        scratch_types=[
            pltpu.VMEM_SHARED((N_NODES, D), jnp.float32),
            pltpu.VMEM((NB, CHUNK), jnp.int32),
            pltpu.VMEM((NB, CHUNK), jnp.int32),
            pltpu.VMEM((NB, CHUNK), jnp.float32),
            pltpu.VMEM((NB, CHUNK), jnp.float32),
            pltpu.VMEM((NB, CHUNK), jnp.int32),
            pltpu.VMEM((CHUNK,), jnp.float32),
            pltpu.VMEM((L,), jnp.float32),
            pltpu.VMEM((NB, CHUNK, D), jnp.float32),
            pltpu.VMEM((ZROWS, D), jnp.float32),
            pltpu.SemaphoreType.DMA((NB,)),
            pltpu.SemaphoreType.DMA((NB,)),
            pltpu.SemaphoreType.DMA((NB,)),
        ],
    )
    def k(x_hbm, src_hbm, dst_hbm, dt_hbm, norm_hbm, lam_hbm, out_hbm,
          acc, srcc, dstc, dtc, normc, dst2, wc, lamv, rows, zbuf,
          msem, gsem, ssem):
        cid = lax.axis_index("c")
        sid = lax.axis_index("s")
        wid = sid * NC + cid
        ebase = wid * E_PER_W

        pltpu.sync_copy(lam_hbm, lamv)
        lamvec = lamv[...]

        def zfill(i, _):
            for k2 in range(D // L):
                zbuf[i, pl.ds(k2 * L, L)] = jnp.zeros((L,), jnp.float32)
            return 0
        lax.fori_loop(0, ZROWS, zfill, 0)
        base_r = sid * WB_ROWS
        for t in range(WB_ROWS // ZROWS):
            pltpu.sync_copy(zbuf, acc.at[pl.ds(base_r + t * ZROWS, ZROWS)])
        @pl.when(sid == 0)
        def _zero_tail():
            pltpu.sync_copy(zbuf, acc.at[pl.ds(NS * WB_ROWS, TAIL_ROWS)])

        plsc.subcore_barrier()

        def start_meta(i, b):
            e0 = ebase + i * CHUNK
            pltpu.async_copy(src_hbm.at[pl.ds(e0, CHUNK)], srcc.at[b],
                             msem.at[b])
            pltpu.async_copy(dst_hbm.at[pl.ds(e0, CHUNK)], dstc.at[b],
                             msem.at[b])
            pltpu.async_copy(dt_hbm.at[pl.ds(e0, CHUNK)], dtc.at[b],
                             msem.at[b])
            pltpu.async_copy(norm_hbm.at[pl.ds(e0, CHUNK)], normc.at[b],
                             msem.at[b])

        def wait_meta(b):
            for ref in (srcc, dstc, dtc, normc):
                pltpu.make_async_copy(src_hbm.at[pl.ds(0, CHUNK)], ref.at[b],
                                      msem.at[b]).wait()

        def start_gather(b):
            pltpu.async_copy(x_hbm.at[srcc.at[b]], rows.at[b], gsem.at[b])

        def wait_gather(b):
            pltpu.make_async_copy(x_hbm.at[srcc.at[b]], rows.at[b],
                                  gsem.at[b]).wait()

        def start_scatter(b):
            pltpu.async_copy(rows.at[b], acc.at[dst2.at[b]], ssem.at[b],
                             add=True)

        def wait_scatter(b):
            pltpu.make_async_copy(rows.at[b], acc.at[dst2.at[b]],
                                  ssem.at[b]).wait()

        def process(i, b, drain, pf1, pf3):
            nb_ = (b + 1) % NB
            if drain:
                wait_scatter(nb_)
            if pf1:
                wait_meta(nb_)
                start_gather(nb_)
            for j2 in range(CHUNK // L):
                sl2 = pl.ds(j2 * L, L)
                dst2[b, sl2] = dstc[b, sl2]
                wc[sl2] = normc[b, sl2] * jnp.exp(-(lamvec * dtc[b, sl2]))
            wait_gather(b)
            if pf3:
                start_meta(i + NB, b)
            def scale_body(j, _):
                wvec = wc[pl.ds(j * L, L)]
                for t in range(L):
                    e = j * L + t
                    ws = wvec[t]
                    for k2 in range(D // L):
                        sl = pl.ds(k2 * L, L)
                        rows[b, e, sl] = rows[b, e, sl] * ws
                return 0
            lax.fori_loop(0, CHUNK // L, scale_body, 0)
            start_scatter(b)

        start_meta(0, 0)
        start_meta(1, 1)
        start_meta(2, 2)
        start_meta(3, 3)
        wait_meta(0)
        start_gather(0)

        process(0, 0, False, True, True)
        process(1, 1, False, True, True)
        process(2, 2, False, True, True)
        process(3, 3, True, True, True)
        def quad_body(p, _):
            i = p * NB
            process(i, 0, True, True, True)
            process(i + 1, 1, True, True, True)
            process(i + 2, 2, True, True, True)
            process(i + 3, 3, True, True, True)
            return 0
        lax.fori_loop(1, 30, quad_body, 0)
        process(120, 0, True, True, True)
        process(121, 1, True, True, False)
        process(122, 2, True, True, False)
        process(123, 3, True, True, False)
        process(124, 0, True, False, False)
        wait_scatter(2)
        wait_scatter(3)
        wait_scatter(0)

        plsc.subcore_barrier()

        pltpu.sync_copy(acc.at[pl.ds(base_r, WB_ROWS)],
                        out_hbm.at[cid, pl.ds(base_r, WB_ROWS)])
        @pl.when(sid == 0)
        def _write_tail():
            pltpu.sync_copy(acc.at[pl.ds(NS * WB_ROWS, TAIL_ROWS)],
                            out_hbm.at[cid, pl.ds(NS * WB_ROWS, TAIL_ROWS)])

    return k(x, src1, dst1, dt1, norm1, lam16)


def _combine(a, b):
    def body(a_ref, b_ref, o_ref):
        o_ref[...] = a_ref[...] + b_ref[...]
    return pl.pallas_call(
        body,
        out_shape=jax.ShapeDtypeStruct((N_NODES, D), jnp.float32),
    )(a, b)


def kernel(x, edge_index, dt, norm, decay_lam):
    src = edge_index[0].astype(jnp.int32)
    dst = edge_index[1].astype(jnp.int32)
    lam = jnp.maximum(decay_lam.astype(jnp.float32), 0.0) + 0.0001
    lam16 = jnp.full((L,), lam, jnp.float32)
    parts = _sc_segment_sum(x, src, dst, dt.astype(jnp.float32),
                            norm.astype(jnp.float32), lam16)
    return _combine(parts[0], parts[1])


# speedup vs baseline: 1.0086x; 1.0086x over previous
import functools

import jax
import jax.numpy as jnp
from jax import lax
from jax.experimental import pallas as pl
from jax.experimental.pallas import tpu as pltpu
from jax.experimental.pallas import tpu_sc as plsc

N_NODES = 10000
D = 128
E = 320000
NC = 2
NS = 16
NW = NC * NS
E_PER_W = E // NW
CHUNK = 80
N_CHUNKS = E_PER_W // CHUNK
NB = 3
WB_ROWS = 624
TAIL_ROWS = N_NODES - NS * WB_ROWS
ZROWS = 16
L = 16


def _sc_segment_sum(x, src1, dst1, dt1, norm1, lam16):
    mesh = plsc.VectorSubcoreMesh(core_axis_name="c", subcore_axis_name="s")

    @functools.partial(
        pl.kernel,
        out_type=jax.ShapeDtypeStruct((NC, N_NODES, D), jnp.float32),
        mesh=mesh,
        scratch_types=[
            pltpu.VMEM_SHARED((N_NODES, D), jnp.float32),
            pltpu.VMEM((NB, CHUNK), jnp.int32),
            pltpu.VMEM((NB, CHUNK), jnp.int32),
            pltpu.VMEM((NB, CHUNK), jnp.float32),
            pltpu.VMEM((NB, CHUNK), jnp.float32),
            pltpu.VMEM((NB, CHUNK), jnp.int32),
            pltpu.VMEM((CHUNK,), jnp.float32),
            pltpu.VMEM((L,), jnp.float32),
            pltpu.VMEM((NB, CHUNK, D), jnp.float32),
            pltpu.VMEM((ZROWS, D), jnp.float32),
            pltpu.SemaphoreType.DMA((NB,)),
            pltpu.SemaphoreType.DMA((NB,)),
            pltpu.SemaphoreType.DMA((NB,)),
        ],
    )
    def k(x_hbm, src_hbm, dst_hbm, dt_hbm, norm_hbm, lam_hbm, out_hbm,
          acc, srcc, dstc, dtc, normc, dst2, wc, lamv, rows, zbuf,
          msem, gsem, ssem):
        cid = lax.axis_index("c")
        sid = lax.axis_index("s")
        wid = sid * NC + cid
        ebase = wid * E_PER_W

        pltpu.sync_copy(lam_hbm, lamv)
        lamvec = lamv[...]

        def zfill(i, _):
            for k2 in range(D // L):
                zbuf[i, pl.ds(k2 * L, L)] = jnp.zeros((L,), jnp.float32)
            return 0
        lax.fori_loop(0, ZROWS, zfill, 0)
        base_r = sid * WB_ROWS
        for t in range(WB_ROWS // ZROWS):
            pltpu.sync_copy(zbuf, acc.at[pl.ds(base_r + t * ZROWS, ZROWS)])
        @pl.when(sid == 0)
        def _zero_tail():
            pltpu.sync_copy(zbuf, acc.at[pl.ds(NS * WB_ROWS, TAIL_ROWS)])

        plsc.subcore_barrier()

        def start_meta(i, b):
            e0 = ebase + i * CHUNK
            pltpu.async_copy(src_hbm.at[pl.ds(e0, CHUNK)], srcc.at[b],
                             msem.at[b])
            pltpu.async_copy(dst_hbm.at[pl.ds(e0, CHUNK)], dstc.at[b],
                             msem.at[b])
            pltpu.async_copy(dt_hbm.at[pl.ds(e0, CHUNK)], dtc.at[b],
                             msem.at[b])
            pltpu.async_copy(norm_hbm.at[pl.ds(e0, CHUNK)], normc.at[b],
                             msem.at[b])

        def wait_meta(b):
            for ref in (srcc, dstc, dtc, normc):
                pltpu.make_async_copy(src_hbm.at[pl.ds(0, CHUNK)], ref.at[b],
                                      msem.at[b]).wait()

        def start_gather(b):
            pltpu.async_copy(x_hbm.at[srcc.at[b]], rows.at[b], gsem.at[b])

        def wait_gather(b):
            pltpu.make_async_copy(x_hbm.at[srcc.at[b]], rows.at[b],
                                  gsem.at[b]).wait()

        def start_scatter(b):
            pltpu.async_copy(rows.at[b], acc.at[dst2.at[b]], ssem.at[b],
                             add=True)

        def wait_scatter(b):
            pltpu.make_async_copy(rows.at[b], acc.at[dst2.at[b]],
                                  ssem.at[b]).wait()

        def process(i, b, drain, pf1, pf3):
            nb_ = (b + 1) % NB
            if drain:
                wait_scatter(nb_)
            if pf1:
                wait_meta(nb_)
                start_gather(nb_)
            for j2 in range(CHUNK // L):
                sl2 = pl.ds(j2 * L, L)
                dst2[b, sl2] = dstc[b, sl2]
                wc[sl2] = normc[b, sl2] * jnp.exp(-(lamvec * dtc[b, sl2]))
            wait_gather(b)
            if pf3:
                start_meta(i + NB, b)
            def scale_body(j, _):
                wvec = wc[pl.ds(j * L, L)]
                for t in range(L):
                    e = j * L + t
                    ws = wvec[t]
                    for k2 in range(D // L):
                        sl = pl.ds(k2 * L, L)
                        rows[b, e, sl] = rows[b, e, sl] * ws
                return 0
            lax.fori_loop(0, CHUNK // L, scale_body, 0)
            start_scatter(b)

        start_meta(0, 0)
        start_meta(1, 1)
        start_meta(2, 2)
        wait_meta(0)
        start_gather(0)

        process(0, 0, False, True, True)
        process(1, 1, False, True, True)
        process(2, 2, True, True, True)
        def triple_body(p, _):
            i = p * NB
            process(i, 0, True, True, True)
            process(i + 1, 1, True, True, True)
            process(i + 2, 2, True, True, True)
            return 0
        lax.fori_loop(1, 40, triple_body, 0)
        process(120, 0, True, True, True)
        process(121, 1, True, True, True)
        process(122, 2, True, True, False)
        process(123, 0, True, True, False)
        process(124, 1, True, False, False)
        wait_scatter(0)
        wait_scatter(1)

        plsc.subcore_barrier()

        pltpu.sync_copy(acc.at[pl.ds(base_r, WB_ROWS)],
                        out_hbm.at[cid, pl.ds(base_r, WB_ROWS)])
        @pl.when(sid == 0)
        def _write_tail():
            pltpu.sync_copy(acc.at[pl.ds(NS * WB_ROWS, TAIL_ROWS)],
                            out_hbm.at[cid, pl.ds(NS * WB_ROWS, TAIL_ROWS)])

    return k(x, src1, dst1, dt1, norm1, lam16)


def _combine(a, b):
    def body(a_ref, b_ref, o_ref):
        o_ref[...] = a_ref[...] + b_ref[...]
    return pl.pallas_call(
        body,
        out_shape=jax.ShapeDtypeStruct((N_NODES, D), jnp.float32),
    )(a, b)


def kernel(x, edge_index, dt, norm, decay_lam):
    src = edge_index[0].astype(jnp.int32)
    dst = edge_index[1].astype(jnp.int32)
    lam = jnp.maximum(decay_lam.astype(jnp.float32), 0.0) + 0.0001
    lam16 = jnp.full((L,), lam, jnp.float32)
    parts = _sc_segment_sum(x, src, dst, dt.astype(jnp.float32),
                            norm.astype(jnp.float32), lam16)
    return _combine(parts[0], parts[1])


# speedup vs baseline: 1.0529x; 1.0440x over previous
import functools

import jax
import jax.numpy as jnp
from jax import lax
from jax.experimental import pallas as pl
from jax.experimental.pallas import tpu as pltpu
from jax.experimental.pallas import tpu_sc as plsc

N_NODES = 10000
D = 128
E = 320000
NC = 2
NS = 16
NW = NC * NS
E_PER_W = E // NW
CHUNK = 80
N_CHUNKS = E_PER_W // CHUNK
NB = 3
WB_ROWS = 624
TAIL_ROWS = N_NODES - NS * WB_ROWS
ZROWS = 16
L = 16


def _sc_segment_sum(x, ei1, dt1, norm1, lam16):
    mesh = plsc.VectorSubcoreMesh(core_axis_name="c", subcore_axis_name="s")

    @functools.partial(
        pl.kernel,
        out_type=jax.ShapeDtypeStruct((NC, N_NODES, D), jnp.float32),
        mesh=mesh,
        scratch_types=[
            pltpu.VMEM_SHARED((N_NODES, D), jnp.float32),
            pltpu.VMEM((NB, CHUNK), jnp.int32),
            pltpu.VMEM((NB, CHUNK), jnp.int32),
            pltpu.VMEM((NB, CHUNK), jnp.float32),
            pltpu.VMEM((NB, CHUNK), jnp.float32),
            pltpu.VMEM((NB, CHUNK), jnp.int32),
            pltpu.VMEM((CHUNK,), jnp.float32),
            pltpu.VMEM((L,), jnp.float32),
            pltpu.VMEM((NB, CHUNK, D), jnp.float32),
            pltpu.VMEM((ZROWS, D), jnp.float32),
            pltpu.VMEM((4 * CHUNK,), jnp.int32),
            pltpu.SemaphoreType.DMA((NB,)),
            pltpu.SemaphoreType.DMA((NB,)),
            pltpu.SemaphoreType.DMA((NB,)),
        ],
    )
    def k(x_hbm, ei_hbm, dt_hbm, norm_hbm, lam_hbm, out_hbm,
          acc, srcc, dstc, dtc, normc, dst2, wc, lamv, rows, zbuf, mdrain,
          msem, gsem, ssem):
        cid = lax.axis_index("c")
        sid = lax.axis_index("s")
        wid = sid * NC + cid
        ebase = wid * E_PER_W

        pltpu.sync_copy(lam_hbm, lamv)
        lamvec = lamv[...]

        def zfill(i, _):
            for k2 in range(D // L):
                zbuf[i, pl.ds(k2 * L, L)] = jnp.zeros((L,), jnp.float32)
            return 0
        lax.fori_loop(0, ZROWS, zfill, 0)
        base_r = sid * WB_ROWS
        for t in range(WB_ROWS // ZROWS):
            pltpu.sync_copy(zbuf, acc.at[pl.ds(base_r + t * ZROWS, ZROWS)])
        @pl.when(sid == 0)
        def _zero_tail():
            pltpu.sync_copy(zbuf, acc.at[pl.ds(NS * WB_ROWS, TAIL_ROWS)])

        plsc.subcore_barrier()

        def start_meta(i, b):
            e0 = ebase + i * CHUNK
            pltpu.async_copy(ei_hbm.at[pl.ds(e0, CHUNK)], srcc.at[b],
                             msem.at[b])
            pltpu.async_copy(ei_hbm.at[pl.ds(E + e0, CHUNK)], dstc.at[b],
                             msem.at[b])
            pltpu.async_copy(dt_hbm.at[pl.ds(e0, CHUNK)], dtc.at[b],
                             msem.at[b])
            pltpu.async_copy(norm_hbm.at[pl.ds(e0, CHUNK)], normc.at[b],
                             msem.at[b])

        def wait_meta(b):
            pltpu.make_async_copy(ei_hbm.at[pl.ds(0, 4 * CHUNK)], mdrain,
                                  msem.at[b]).wait()

        def start_gather(b):
            pltpu.async_copy(x_hbm.at[srcc.at[b]], rows.at[b], gsem.at[b])

        def wait_gather(b):
            pltpu.make_async_copy(x_hbm.at[srcc.at[b]], rows.at[b],
                                  gsem.at[b]).wait()

        def start_scatter(b):
            pltpu.async_copy(rows.at[b], acc.at[dst2.at[b]], ssem.at[b],
                             add=True)

        def wait_scatter(b):
            pltpu.make_async_copy(rows.at[b], acc.at[dst2.at[b]],
                                  ssem.at[b]).wait()

        def process(i, b, drain, pf1, pf3):
            nb_ = (b + 1) % NB
            if drain:
                wait_scatter(nb_)
            if pf1:
                wait_meta(nb_)
                start_gather(nb_)
            for j2 in range(CHUNK // L):
                sl2 = pl.ds(j2 * L, L)
                dst2[b, sl2] = dstc[b, sl2]
                wc[sl2] = normc[b, sl2] * jnp.exp(-(lamvec * dtc[b, sl2]))
            wait_gather(b)
            if pf3:
                start_meta(i + NB, b)
            def scale_body(j, _):
                wvec = wc[pl.ds(j * L, L)]
                for t in range(L):
                    e = j * L + t
                    ws = wvec[t]
                    for k2 in range(D // L):
                        sl = pl.ds(k2 * L, L)
                        rows[b, e, sl] = rows[b, e, sl] * ws
                return 0
            lax.fori_loop(0, CHUNK // L, scale_body, 0)
            start_scatter(b)

        start_meta(0, 0)
        start_meta(1, 1)
        start_meta(2, 2)
        wait_meta(0)
        start_gather(0)

        process(0, 0, False, True, True)
        process(1, 1, False, True, True)
        process(2, 2, True, True, True)
        def triple_body(p, _):
            i = p * NB
            process(i, 0, True, True, True)
            process(i + 1, 1, True, True, True)
            process(i + 2, 2, True, True, True)
            return 0
        lax.fori_loop(1, 40, triple_body, 0)
        process(120, 0, True, True, True)
        process(121, 1, True, True, True)
        process(122, 2, True, True, False)
        process(123, 0, True, True, False)
        process(124, 1, True, False, False)
        wait_scatter(0)
        wait_scatter(1)

        plsc.subcore_barrier()

        pltpu.sync_copy(acc.at[pl.ds(base_r, WB_ROWS)],
                        out_hbm.at[cid, pl.ds(base_r, WB_ROWS)])
        @pl.when(sid == 0)
        def _write_tail():
            pltpu.sync_copy(acc.at[pl.ds(NS * WB_ROWS, TAIL_ROWS)],
                            out_hbm.at[cid, pl.ds(NS * WB_ROWS, TAIL_ROWS)])

    return k(x, ei1, dt1, norm1, lam16)


def _combine(a, b):
    def body(a_ref, b_ref, o_ref):
        o_ref[...] = a_ref[...] + b_ref[...]
    blk = 1000
    return pl.pallas_call(
        body,
        grid=(N_NODES // 1000,),
        in_specs=[pl.BlockSpec((blk, D), lambda i: (i, 0)),
                  pl.BlockSpec((blk, D), lambda i: (i, 0))],
        out_specs=pl.BlockSpec((blk, D), lambda i: (i, 0)),
        out_shape=jax.ShapeDtypeStruct((N_NODES, D), jnp.float32),
    )(a, b)


def kernel(x, edge_index, dt, norm, decay_lam):
    ei1 = edge_index.astype(jnp.int32).reshape(2 * E)
    lam = jnp.maximum(decay_lam.astype(jnp.float32), 0.0) + 0.0001
    lam16 = jnp.full((L,), lam, jnp.float32)
    parts = _sc_segment_sum(x, ei1, dt.astype(jnp.float32),
                            norm.astype(jnp.float32), lam16)
    return _combine(parts[0], parts[1])


# speedup vs baseline: 1.0638x; 1.0104x over previous
import functools

import jax
import jax.numpy as jnp
from jax import lax
from jax.experimental import pallas as pl
from jax.experimental.pallas import tpu as pltpu
from jax.experimental.pallas import tpu_sc as plsc

N_NODES = 10000
D = 128
E = 320000
NC = 2
NS = 16
NW = NC * NS
E_PER_W = E // NW
CHUNK = 80
N_CHUNKS = E_PER_W // CHUNK
NB = 3
WB_ROWS = 624
TAIL_ROWS = N_NODES - NS * WB_ROWS
ZROWS = 48
L = 16


def _sc_segment_sum(x, ei1, dt1, norm1, lam16):
    mesh = plsc.VectorSubcoreMesh(core_axis_name="c", subcore_axis_name="s")

    @functools.partial(
        pl.kernel,
        out_type=jax.ShapeDtypeStruct((NC, N_NODES, D), jnp.float32),
        mesh=mesh,
        scratch_types=[
            pltpu.VMEM_SHARED((N_NODES, D), jnp.float32),
            pltpu.VMEM((NB, CHUNK), jnp.int32),
            pltpu.VMEM((NB, CHUNK), jnp.int32),
            pltpu.VMEM((NB, CHUNK), jnp.float32),
            pltpu.VMEM((NB, CHUNK), jnp.float32),
            pltpu.VMEM((NB, CHUNK), jnp.int32),
            pltpu.VMEM((CHUNK,), jnp.float32),
            pltpu.VMEM((L,), jnp.float32),
            pltpu.VMEM((NB, CHUNK, D), jnp.float32),
            pltpu.VMEM((ZROWS, D), jnp.float32),
            pltpu.VMEM((4 * CHUNK,), jnp.int32),
            pltpu.SemaphoreType.DMA((NB,)),
            pltpu.SemaphoreType.DMA((NB,)),
            pltpu.SemaphoreType.DMA((NB,)),
        ],
    )
    def k(x_hbm, ei_hbm, dt_hbm, norm_hbm, lam_hbm, out_hbm,
          acc, srcc, dstc, dtc, normc, dst2, wc, lamv, rows, zbuf, mdrain,
          msem, gsem, ssem):
        cid = lax.axis_index("c")
        sid = lax.axis_index("s")
        wid = sid * NC + cid
        ebase = wid * E_PER_W

        def start_meta(i, b):
            e0 = ebase + i * CHUNK
            pltpu.async_copy(ei_hbm.at[pl.ds(e0, CHUNK)], srcc.at[b],
                             msem.at[b])
            pltpu.async_copy(ei_hbm.at[pl.ds(E + e0, CHUNK)], dstc.at[b],
                             msem.at[b])
            pltpu.async_copy(dt_hbm.at[pl.ds(e0, CHUNK)], dtc.at[b],
                             msem.at[b])
            pltpu.async_copy(norm_hbm.at[pl.ds(e0, CHUNK)], normc.at[b],
                             msem.at[b])

        start_meta(0, 0)
        start_meta(1, 1)
        start_meta(2, 2)
        pltpu.sync_copy(lam_hbm, lamv)
        lamvec = lamv[...]

        def zfill(i, _):
            for k2 in range(D // L):
                zbuf[i, pl.ds(k2 * L, L)] = jnp.zeros((L,), jnp.float32)
            return 0
        lax.fori_loop(0, ZROWS, zfill, 0)
        base_r = sid * WB_ROWS
        for t in range(WB_ROWS // ZROWS):
            pltpu.sync_copy(zbuf, acc.at[pl.ds(base_r + t * ZROWS, ZROWS)])
        @pl.when(sid == 0)
        def _zero_tail():
            pltpu.sync_copy(zbuf.at[pl.ds(0, TAIL_ROWS)],
                            acc.at[pl.ds(NS * WB_ROWS, TAIL_ROWS)])

        plsc.subcore_barrier()

        def wait_meta(b):
            pltpu.make_async_copy(ei_hbm.at[pl.ds(0, 4 * CHUNK)], mdrain,
                                  msem.at[b]).wait()

        def start_gather(b):
            pltpu.async_copy(x_hbm.at[srcc.at[b]], rows.at[b], gsem.at[b])

        def wait_gather(b):
            pltpu.make_async_copy(x_hbm.at[srcc.at[b]], rows.at[b],
                                  gsem.at[b]).wait()

        def start_scatter(b):
            pltpu.async_copy(rows.at[b], acc.at[dst2.at[b]], ssem.at[b],
                             add=True)

        def wait_scatter(b):
            pltpu.make_async_copy(rows.at[b], acc.at[dst2.at[b]],
                                  ssem.at[b]).wait()

        def process(i, b, drain, pf1, pf3):
            nb_ = (b + 1) % NB
            if drain:
                wait_scatter(nb_)
            if pf1:
                wait_meta(nb_)
                start_gather(nb_)
            for j2 in range(CHUNK // L):
                sl2 = pl.ds(j2 * L, L)
                dst2[b, sl2] = dstc[b, sl2]
                wc[sl2] = normc[b, sl2] * jnp.exp(-(lamvec * dtc[b, sl2]))
            wait_gather(b)
            if pf3:
                start_meta(i + NB, b)
            def scale_body(j, _):
                wvec = wc[pl.ds(j * L, L)]
                for t in range(L):
                    e = j * L + t
                    ws = wvec[t]
                    for k2 in range(D // L):
                        sl = pl.ds(k2 * L, L)
                        rows[b, e, sl] = rows[b, e, sl] * ws
                return 0
            lax.fori_loop(0, CHUNK // L, scale_body, 0)
            start_scatter(b)

        wait_meta(0)
        start_gather(0)

        process(0, 0, False, True, True)
        process(1, 1, False, True, True)
        process(2, 2, True, True, True)
        def triple_body(p, _):
            i = p * NB
            process(i, 0, True, True, True)
            process(i + 1, 1, True, True, True)
            process(i + 2, 2, True, True, True)
            return 0
        lax.fori_loop(1, 40, triple_body, 0)
        process(120, 0, True, True, True)
        process(121, 1, True, True, True)
        process(122, 2, True, True, False)
        process(123, 0, True, True, False)
        process(124, 1, True, False, False)
        wait_scatter(0)
        wait_scatter(1)

        plsc.subcore_barrier()

        pltpu.sync_copy(acc.at[pl.ds(base_r, WB_ROWS)],
                        out_hbm.at[cid, pl.ds(base_r, WB_ROWS)])
        @pl.when(sid == 0)
        def _write_tail():
            pltpu.sync_copy(acc.at[pl.ds(NS * WB_ROWS, TAIL_ROWS)],
                            out_hbm.at[cid, pl.ds(NS * WB_ROWS, TAIL_ROWS)])

    return k(x, ei1, dt1, norm1, lam16)


def _combine(a, b):
    def body(a_ref, b_ref, o_ref):
        o_ref[...] = a_ref[...] + b_ref[...]
    blk = 1000
    return pl.pallas_call(
        body,
        grid=(N_NODES // 1000,),
        in_specs=[pl.BlockSpec((blk, D), lambda i: (i, 0)),
                  pl.BlockSpec((blk, D), lambda i: (i, 0))],
        out_specs=pl.BlockSpec((blk, D), lambda i: (i, 0)),
        out_shape=jax.ShapeDtypeStruct((N_NODES, D), jnp.float32),
    )(a, b)


def kernel(x, edge_index, dt, norm, decay_lam):
    ei1 = edge_index.astype(jnp.int32).reshape(2 * E)
    lam = jnp.maximum(decay_lam.astype(jnp.float32), 0.0) + 0.0001
    lam16 = jnp.full((L,), lam, jnp.float32)
    parts = _sc_segment_sum(x, ei1, dt.astype(jnp.float32),
                            norm.astype(jnp.float32), lam16)
    return _combine(parts[0], parts[1])
